# Initial kernel scaffold; baseline (speedup 1.0000x reference)
import jax
import jax.numpy as jnp
from jax.experimental import pallas as pl


def kernel(node_feats, edge_feats, params, edge_index, node_graph_ids):
    raise NotImplementedError("write your pallas kernel here")


# speedup vs baseline: 8.5875x; 8.5875x over previous
import functools

import jax
import jax.numpy as jnp
from jax import lax
from jax.experimental import pallas as pl
from jax.experimental.pallas import tpu as pltpu
from jax.experimental.pallas import tpu_sc as plsc

V, E, NF, EF, GF, G = 10000, 320000, 128, 16, 64, 256

NC, NS, L = 2, 16, 16
NW = NC * NS
EPW = E // NW
CHUNK = 80
NCH = EPW // CHUNK
ACC_W = 80
VP = 10240
VPT = VP // NS

BV = 2000
BE = 8000
BN = 1000

_F32 = jnp.float32


def _lrelu(x):
    return jnp.maximum(x, 0.01 * x)


def _elu(x):
    return jnp.where(x > 0, x, jnp.exp(jnp.minimum(x, 0.0)) - 1.0)


def _sigmoid(x):
    return 1.0 / (1.0 + jnp.exp(-x))


def _gru(x, h, wih_t, whh_t, bih, bhh):
    gi = jnp.dot(x, wih_t, preferred_element_type=_F32) + bih
    gh = jnp.dot(h, whh_t, preferred_element_type=_F32) + bhh
    r = _sigmoid(gi[:, 0:GF] + gh[:, 0:GF])
    z = _sigmoid(gi[:, GF:2 * GF] + gh[:, GF:2 * GF])
    n = jnp.tanh(gi[:, 2 * GF:] + r * gh[:, 2 * GF:])
    return (1.0 - z) * n + z * h


def tc_prep_body(nf_ref, wpn_ref, bpn_ref, wa_ref, wcb_ref,
                 hv_ref, p_ref, q_ref):
    nf = nf_ref[...]
    hv = _lrelu(jnp.dot(nf, wpn_ref[...], preferred_element_type=_F32)
                + bpn_ref[...])
    hv_ref[...] = hv
    p_ref[...] = jnp.dot(nf, wa_ref[...], preferred_element_type=_F32)
    q_ref[...] = jnp.dot(hv, wcb_ref[...], preferred_element_type=_F32)


def tc_eb_body(ef_ref, wb_ref, bpe1_ref, eb_ref):
    eb_ref[...] = (jnp.dot(ef_ref[...], wb_ref[...],
                           preferred_element_type=_F32) + bpe1_ref[...])


def tc_gc_update_body(acc_ref, hv_ref, wet_ref, bet_ref,
                      wih_ref, whh_ref, bih_ref, bhh_ref,
                      wpn1_ref, bpn1_ref, wuv_ref,
                      h_ref, hp_ref, uwv_ref):
    accs = acc_ref[...]
    asum = accs[0] + accs[1]
    t = asum[:, :GF]
    den = asum[:, GF:GF + 1]
    rec = 1.0 / (den + 1e-12)
    ctx = _elu(jnp.dot(t * rec, wet_ref[...], preferred_element_type=_F32)
               + (den * rec) * bet_ref[...])
    hv = hv_ref[...]
    h = jnp.maximum(_gru(ctx, hv, wih_ref[...], whh_ref[...],
                         bih_ref[...], bhh_ref[...]), 0.0)
    h_ref[...] = h
    hp_ref[...] = (jnp.dot(h, wpn1_ref[...], preferred_element_type=_F32)
                   + bpn1_ref[...])
    uwv_ref[...] = jnp.dot(h, wuv_ref[...], preferred_element_type=_F32)


def tc_l1_update_body(acc_ref, h_ref, wih_ref, whh_ref, bih_ref, bhh_ref,
                      wpn0_ref, bpn0_ref, wpn1_ref, bpn1_ref,
                      wc2_ref, bc2_ref,
                      h2_ref, hvp0_ref, hvp1_ref, c2_ref):
    accs = acc_ref[...]
    asum = accs[0] + accs[1]
    t = asum[:, :GF]
    den = asum[:, GF:GF + 1]
    ctx = _elu(t / (den + 1e-12))
    h = h_ref[...]
    h2 = jnp.maximum(_gru(ctx, h, wih_ref[...], whh_ref[...],
                          bih_ref[...], bhh_ref[...]), 0.0)
    h2_ref[...] = h2
    hvp0_ref[...] = (jnp.dot(h2, wpn0_ref[...], preferred_element_type=_F32)
                     + bpn0_ref[...])
    hvp1_ref[...] = (jnp.dot(h2, wpn1_ref[...], preferred_element_type=_F32)
                     + bpn1_ref[...])
    c2_ref[...] = (jnp.dot(h2, wc2_ref[...], preferred_element_type=_F32)
                   + bc2_ref[...])


def tc_readout_body(h2_ref, hvp0_ref, hvp1_ref, c2_ref, gidf_ref,
                    wca0_ref, wca1_ref,
                    wih0_ref, whh0_ref, bih0_ref, bhh0_ref,
                    wih1_ref, whh1_ref, bih1_ref, bhh1_ref,
                    out_ref):
    nblk = V // BN
    giota = lax.broadcasted_iota(jnp.int32, (G, BN), 0).astype(_F32)

    def onehot(vb):
        gb = gidf_ref[pl.ds(vb, 1), :]
        return (giota == gb).astype(_F32)

    g = jnp.zeros((G, GF), _F32)
    for vb in range(nblk):
        g = g + jnp.dot(onehot(vb), h2_ref[pl.ds(vb * BN, BN), :],
                        preferred_element_type=_F32)

    for r in range(2):
        wca = (wca0_ref, wca1_ref)[r][...]
        hvp_ref = (hvp0_ref, hvp1_ref)[r]
        rg = jnp.maximum(g, 0.0)
        s1 = jnp.dot(rg, wca, preferred_element_type=_F32)
        tacc = jnp.zeros((G, GF), _F32)
        dacc = jnp.zeros((G, 8), _F32)
        for vb in range(nblk):
            oh = onehot(vb)
            s1n = lax.dot_general(oh, s1, (((0,), (0,)), ((), ())),
                                  preferred_element_type=_F32)
            c2b = c2_ref[pl.ds(vb * BN, BN), r * GF:r * GF + 1]
            w = jnp.exp(_lrelu(s1n[:, 0:1] + c2b))
            hvpb = hvp_ref[pl.ds(vb * BN, BN), :]
            tacc = tacc + jnp.dot(oh, w * hvpb,
                                  preferred_element_type=_F32)
            dacc = dacc + jnp.dot(oh, jnp.broadcast_to(w, (BN, 8)),
                                  preferred_element_type=_F32)
        ctx = _elu(tacc / (dacc[:, 0:1] + 1e-12))
        wih = (wih0_ref, wih1_ref)[r][...]
        whh = (whh0_ref, whh1_ref)[r][...]
        bih = (bih0_ref, bih1_ref)[r][...]
        bhh = (bhh0_ref, bhh1_ref)[r][...]
        g = jnp.maximum(_gru(ctx, g, wih, whh, bih, bhh), 0.0)
    out_ref[...] = g


_SC_MESH = dict(core_axis_name="c", subcore_axis_name="s",
                num_cores=NC, num_subcores=NS)


def sc_gc_body(p_hbm, eb_hbm, q_hbm, wd_hbm, src_hbm, dst_hbm, zero_hbm,
               out_hbm, sidx, didx, prow, ebrow, msg, qv, wdv, acc, sem):
    cid = lax.axis_index("c")
    sid = lax.axis_index("s")
    wid = cid * NS + sid

    pltpu.sync_copy(zero_hbm.at[pl.ds(sid * VPT, VPT)],
                    acc.at[pl.ds(sid * VPT, VPT)])
    pltpu.sync_copy(q_hbm, qv)
    pltpu.sync_copy(wd_hbm, wdv)
    wd = [wdv[pl.ds(k * L, L)] for k in range(GF // L)]
    b2 = wdv[pl.ds(GF, L)][0]
    plsc.subcore_barrier()

    def chunk_body(c, _):
        base = wid * EPW + c * CHUNK
        pltpu.sync_copy(src_hbm.at[pl.ds(base, CHUNK)], sidx)
        pltpu.sync_copy(dst_hbm.at[pl.ds(base, CHUNK)], didx)
        pltpu.async_copy(p_hbm.at[sidx], prow, sem).wait()
        pltpu.sync_copy(eb_hbm.at[pl.ds(base, CHUNK)], ebrow)

        def group_body(g, _):
            dv = didx[pl.ds(g * L, L)]
            qd = plsc.load_gather(qv, [dv])
            for e in range(L):
                i = g * L + e
                hrows = []
                t = jnp.zeros((L,), _F32)
                for k in range(GF // L):
                    s = prow[i, pl.ds(k * L, L)] + ebrow[i, pl.ds(k * L, L)]
                    hk = jnp.maximum(s, 0.01 * s)
                    hrows.append(hk)
                    t = t + hk * wd[k]
                lg = qd[e] + jnp.sum(t) + b2
                lg = jnp.maximum(lg, 0.01 * lg)
                w = jnp.exp(jnp.full((L,), lg, _F32))
                for k in range(GF // L):
                    msg[i, pl.ds(k * L, L)] = hrows[k] * w
                msg[i, pl.ds(GF, L)] = w
            return 0

        lax.fori_loop(0, CHUNK // L, group_body, 0)
        pltpu.sync_copy(msg, acc.at[didx], add=True)
        return 0

    lax.fori_loop(0, NCH, chunk_body, 0)
    plsc.subcore_barrier()
    pltpu.sync_copy(acc.at[pl.ds(sid * VPT, VPT)],
                    out_hbm.at[cid].at[pl.ds(sid * VPT, VPT)])


def sc_l1_body(hp_hbm, u_hbm, wv_hbm, b_hbm, src_hbm, dst_hbm, zero_hbm,
               out_hbm, sidx, didx, hprow, msg, uv, wvv, bv, acc, sem):
    cid = lax.axis_index("c")
    sid = lax.axis_index("s")
    wid = cid * NS + sid

    pltpu.sync_copy(zero_hbm.at[pl.ds(sid * VPT, VPT)],
                    acc.at[pl.ds(sid * VPT, VPT)])
    pltpu.sync_copy(u_hbm, uv)
    pltpu.sync_copy(wv_hbm, wvv)
    pltpu.sync_copy(b_hbm, bv)
    bl = bv[...][0]
    plsc.subcore_barrier()

    def chunk_body(c, _):
        base = wid * EPW + c * CHUNK
        pltpu.sync_copy(src_hbm.at[pl.ds(base, CHUNK)], sidx)
        pltpu.sync_copy(dst_hbm.at[pl.ds(base, CHUNK)], didx)
        pltpu.async_copy(hp_hbm.at[sidx], hprow, sem).wait()

        def group_body(g, _):
            dv = didx[pl.ds(g * L, L)]
            sv = sidx[pl.ds(g * L, L)]
            lg = plsc.load_gather(uv, [dv]) + plsc.load_gather(wvv, [sv]) + bl
            lg = jnp.maximum(lg, 0.01 * lg)
            wvec = jnp.exp(lg)
            for e in range(L):
                i = g * L + e
                w = jnp.full((L,), wvec[e], _F32)
                for k in range(GF // L):
                    msg[i, pl.ds(k * L, L)] = hprow[i, pl.ds(k * L, L)] * w
                msg[i, pl.ds(GF, L)] = w
            return 0

        lax.fori_loop(0, CHUNK // L, group_body, 0)
        pltpu.sync_copy(msg, acc.at[didx], add=True)
        return 0

    lax.fori_loop(0, NCH, chunk_body, 0)
    plsc.subcore_barrier()
    pltpu.sync_copy(acc.at[pl.ds(sid * VPT, VPT)],
                    out_hbm.at[cid].at[pl.ds(sid * VPT, VPT)])


def _full_spec(shape):
    nd = len(shape)
    return pl.BlockSpec(shape, lambda i, _n=nd: (0,) * _n)


def _call_tc_prep(nf, wpn, bpn, wa, wcb):
    return pl.pallas_call(
        tc_prep_body,
        grid=(V // BV,),
        in_specs=[
            pl.BlockSpec((BV, NF), lambda i: (i, 0)),
            _full_spec(wpn.shape), _full_spec(bpn.shape),
            _full_spec(wa.shape), _full_spec(wcb.shape),
        ],
        out_specs=[
            pl.BlockSpec((BV, GF), lambda i: (i, 0)),
            pl.BlockSpec((BV, GF), lambda i: (i, 0)),
            pl.BlockSpec((BV, 128), lambda i: (i, 0)),
        ],
        out_shape=[
            jax.ShapeDtypeStruct((V, GF), _F32),
            jax.ShapeDtypeStruct((V, GF), _F32),
            jax.ShapeDtypeStruct((V, 128), _F32),
        ],
    )(nf, wpn, bpn, wa, wcb)


def _call_tc_eb(ef, wb, bpe1):
    return pl.pallas_call(
        tc_eb_body,
        grid=(E // BE,),
        in_specs=[
            pl.BlockSpec((BE, EF), lambda i: (i, 0)),
            _full_spec(wb.shape), _full_spec(bpe1.shape),
        ],
        out_specs=pl.BlockSpec((BE, GF), lambda i: (i, 0)),
        out_shape=jax.ShapeDtypeStruct((E, GF), _F32),
    )(ef, wb, bpe1)


def _call_tc_gc_update(acc, hv, wet, bet, wih, whh, bih, bhh,
                       wpn1, bpn1, wuv):
    return pl.pallas_call(
        tc_gc_update_body,
        grid=(V // BV,),
        in_specs=[
            pl.BlockSpec((NC, BV, ACC_W), lambda i: (0, i, 0)),
            pl.BlockSpec((BV, GF), lambda i: (i, 0)),
            _full_spec(wet.shape), _full_spec(bet.shape),
            _full_spec(wih.shape), _full_spec(whh.shape),
            _full_spec(bih.shape), _full_spec(bhh.shape),
            _full_spec(wpn1.shape), _full_spec(bpn1.shape),
            _full_spec(wuv.shape),
        ],
        out_specs=[
            pl.BlockSpec((BV, GF), lambda i: (i, 0)),
            pl.BlockSpec((BV, GF), lambda i: (i, 0)),
            pl.BlockSpec((BV, 128), lambda i: (i, 0)),
        ],
        out_shape=[
            jax.ShapeDtypeStruct((V, GF), _F32),
            jax.ShapeDtypeStruct((V, GF), _F32),
            jax.ShapeDtypeStruct((V, 128), _F32),
        ],
    )(acc, hv, wet, bet, wih, whh, bih, bhh, wpn1, bpn1, wuv)


def _call_tc_l1_update(acc, h, wih, whh, bih, bhh,
                       wpn0, bpn0, wpn1, bpn1, wc2, bc2):
    return pl.pallas_call(
        tc_l1_update_body,
        grid=(V // BV,),
        in_specs=[
            pl.BlockSpec((NC, BV, ACC_W), lambda i: (0, i, 0)),
            pl.BlockSpec((BV, GF), lambda i: (i, 0)),
            _full_spec(wih.shape), _full_spec(whh.shape),
            _full_spec(bih.shape), _full_spec(bhh.shape),
            _full_spec(wpn0.shape), _full_spec(bpn0.shape),
            _full_spec(wpn1.shape), _full_spec(bpn1.shape),
            _full_spec(wc2.shape), _full_spec(bc2.shape),
        ],
        out_specs=[
            pl.BlockSpec((BV, GF), lambda i: (i, 0)),
            pl.BlockSpec((BV, GF), lambda i: (i, 0)),
            pl.BlockSpec((BV, GF), lambda i: (i, 0)),
            pl.BlockSpec((BV, 128), lambda i: (i, 0)),
        ],
        out_shape=[
            jax.ShapeDtypeStruct((V, GF), _F32),
            jax.ShapeDtypeStruct((V, GF), _F32),
            jax.ShapeDtypeStruct((V, GF), _F32),
            jax.ShapeDtypeStruct((V, 128), _F32),
        ],
    )(acc, h, wih, whh, bih, bhh, wpn0, bpn0, wpn1, bpn1, wc2, bc2)


def _call_tc_readout(h2, hvp0, hvp1, c2, gidf, wca0, wca1,
                     wih0, whh0, bih0, bhh0, wih1, whh1, bih1, bhh1):
    return pl.pallas_call(
        tc_readout_body,
        out_shape=jax.ShapeDtypeStruct((G, GF), _F32),
    )(h2, hvp0, hvp1, c2, gidf, wca0, wca1,
      wih0, whh0, bih0, bhh0, wih1, whh1, bih1, bhh1)


def _call_sc_gc(p, eb, q, wdpack, src, dst, zeros):
    f = functools.partial(
        pl.kernel,
        out_type=jax.ShapeDtypeStruct((NC, VP, ACC_W), _F32),
        mesh=plsc.VectorSubcoreMesh(**_SC_MESH),
        compiler_params=pltpu.CompilerParams(needs_layout_passes=False, use_tc_tiling_on_sc=False),
        scratch_types=[
            pltpu.VMEM((CHUNK,), jnp.int32),
            pltpu.VMEM((CHUNK,), jnp.int32),
            pltpu.VMEM((CHUNK, GF), _F32),
            pltpu.VMEM((CHUNK, GF), _F32),
            pltpu.VMEM((CHUNK, ACC_W), _F32),
            pltpu.VMEM((V,), _F32),
            pltpu.VMEM((ACC_W,), _F32),
            pltpu.VMEM_SHARED((VP, ACC_W), _F32),
            pltpu.SemaphoreType.DMA,
        ],
    )(sc_gc_body)
    return f(p, eb, q, wdpack, src, dst, zeros)


def _call_sc_l1(hp, u, wv, bpack, src, dst, zeros):
    f = functools.partial(
        pl.kernel,
        out_type=jax.ShapeDtypeStruct((NC, VP, ACC_W), _F32),
        mesh=plsc.VectorSubcoreMesh(**_SC_MESH),
        compiler_params=pltpu.CompilerParams(needs_layout_passes=False, use_tc_tiling_on_sc=False),
        scratch_types=[
            pltpu.VMEM((CHUNK,), jnp.int32),
            pltpu.VMEM((CHUNK,), jnp.int32),
            pltpu.VMEM((CHUNK, GF), _F32),
            pltpu.VMEM((CHUNK, ACC_W), _F32),
            pltpu.VMEM((V,), _F32),
            pltpu.VMEM((V,), _F32),
            pltpu.VMEM((L,), _F32),
            pltpu.VMEM_SHARED((VP, ACC_W), _F32),
            pltpu.SemaphoreType.DMA,
        ],
    )(sc_l1_body)
    return f(hp, u, wv, bpack, src, dst, zeros)


def kernel(node_feats, edge_feats, params, edge_index, node_graph_ids):
    p_ = params
    src = edge_index[0]
    dst = edge_index[1]

    wpn = p_["gc_pn"]["W"]
    bpn = p_["gc_pn"]["b"].reshape(1, GF)
    wpe1 = p_["gc_pe1"]["W"]
    wa = wpe1[:NF]
    wb = wpe1[NF:]
    bpe1 = p_["gc_pe1"]["b"].reshape(1, GF)
    wpe2 = p_["gc_pe2"]["W"][:, 0]
    bpe2 = p_["gc_pe2"]["b"][0]
    wc = wpe2[:GF]
    wd = wpe2[GF:]
    wcb = jnp.broadcast_to(wc[:, None], (GF, 128))
    wdpack = jnp.zeros((ACC_W,), _F32).at[:GF].set(wd).at[GF].set(bpe2)

    zeros_acc = jnp.zeros((VP, ACC_W), _F32)

    hv, p, qpad = _call_tc_prep(node_feats, wpn, bpn, wa, wcb)
    q = qpad[:, 0]
    eb = _call_tc_eb(edge_feats, wb, bpe1)

    acc_gc = _call_sc_gc(p, eb, q, wdpack, src, dst, zeros_acc)

    wet = p_["gc_et"]["W"]
    bet = p_["gc_et"]["b"].reshape(1, GF)
    g_gru = p_["gc_gru"]
    wl1 = p_["l1_pe"]["W"][:, 0]
    bl1 = p_["l1_pe"]["b"][0]
    wuv = jnp.concatenate([
        jnp.broadcast_to(wl1[:GF, None], (GF, 64)),
        jnp.broadcast_to(wl1[GF:, None], (GF, 64)),
    ], axis=1)
    h, hp, uwv = _call_tc_gc_update(
        acc_gc, hv, wet, bet,
        g_gru["Wih"].T, g_gru["Whh"].T,
        g_gru["bih"].reshape(1, 3 * GF), g_gru["bhh"].reshape(1, 3 * GF),
        p_["l1_pn"]["W"], p_["l1_pn"]["b"].reshape(1, GF), wuv)
    u = uwv[:, 0]
    wv = uwv[:, 64]
    bpack = jnp.full((L,), bl1, _F32)

    acc_l1 = _call_sc_l1(hp, u, wv, bpack, src, dst, zeros_acc)

    l_gru = p_["l1_gru"]
    wc0 = p_["r0_cl"]["W"][:, 0]
    bc0 = p_["r0_cl"]["b"][0]
    wc1 = p_["r1_cl"]["W"][:, 0]
    bc1 = p_["r1_cl"]["b"][0]
    wc2 = jnp.concatenate([
        jnp.broadcast_to(wc0[GF:, None], (GF, 64)),
        jnp.broadcast_to(wc1[GF:, None], (GF, 64)),
    ], axis=1)
    bc2 = jnp.concatenate([jnp.full((1, 64), bc0, _F32),
                           jnp.full((1, 64), bc1, _F32)], axis=1)
    h2, hvp0, hvp1, c2 = _call_tc_l1_update(
        acc_l1, h,
        l_gru["Wih"].T, l_gru["Whh"].T,
        l_gru["bih"].reshape(1, 3 * GF), l_gru["bhh"].reshape(1, 3 * GF),
        p_["r0_pn"]["W"], p_["r0_pn"]["b"].reshape(1, GF),
        p_["r1_pn"]["W"], p_["r1_pn"]["b"].reshape(1, GF),
        wc2, bc2)

    gidf = node_graph_ids.astype(_F32).reshape(V // BN, BN)
    wca0 = jnp.broadcast_to(wc0[:GF, None], (GF, 8))
    wca1 = jnp.broadcast_to(wc1[:GF, None], (GF, 8))
    r0, r1 = p_["r0_gru"], p_["r1_gru"]
    out = _call_tc_readout(
        h2, hvp0, hvp1, c2, gidf, wca0, wca1,
        r0["Wih"].T, r0["Whh"].T,
        r0["bih"].reshape(1, 3 * GF), r0["bhh"].reshape(1, 3 * GF),
        r1["Wih"].T, r1["Whh"].T,
        r1["bih"].reshape(1, 3 * GF), r1["bhh"].reshape(1, 3 * GF))
    return out


# speedup vs baseline: 12.1454x; 1.4143x over previous
import functools

import jax
import jax.numpy as jnp
from jax import lax
from jax.experimental import pallas as pl
from jax.experimental.pallas import tpu as pltpu
from jax.experimental.pallas import tpu_sc as plsc

V, E, NF, EF, GF, G = 10000, 320000, 128, 16, 64, 256

NC, NS, L = 2, 16, 16
NW = NC * NS
EPW = E // NW
CHUNK = 80
NCH = EPW // CHUNK
ACC_W = 80
VP = 10240
VPT = VP // NS

BV = 2000
BE = 8000
BN = 1000

_F32 = jnp.float32


def _lrelu(x):
    return jnp.maximum(x, 0.01 * x)


def _elu(x):
    return jnp.where(x > 0, x, jnp.exp(jnp.minimum(x, 0.0)) - 1.0)


def _sigmoid(x):
    return 1.0 / (1.0 + jnp.exp(-x))


def _gru(x, h, wih_t, whh_t, bih, bhh):
    gi = jnp.dot(x, wih_t, preferred_element_type=_F32) + bih
    gh = jnp.dot(h, whh_t, preferred_element_type=_F32) + bhh
    r = _sigmoid(gi[:, 0:GF] + gh[:, 0:GF])
    z = _sigmoid(gi[:, GF:2 * GF] + gh[:, GF:2 * GF])
    n = jnp.tanh(gi[:, 2 * GF:] + r * gh[:, 2 * GF:])
    return (1.0 - z) * n + z * h


def tc_prep_body(nf_ref, wpn_ref, bpn_ref, wa_ref, wcb_ref,
                 hv_ref, p_ref, q_ref):
    nf = nf_ref[...]
    hv = _lrelu(jnp.dot(nf, wpn_ref[...], preferred_element_type=_F32)
                + bpn_ref[...])
    hv_ref[...] = hv
    p_ref[...] = jnp.dot(nf, wa_ref[...], preferred_element_type=_F32)
    q_ref[...] = jnp.dot(hv, wcb_ref[...], preferred_element_type=_F32)


def tc_eb_body(ef_ref, wb_ref, bpe1_ref, eb_ref):
    eb_ref[...] = (jnp.dot(ef_ref[...], wb_ref[...],
                           preferred_element_type=_F32) + bpe1_ref[...])


def tc_gc_update_body(acc_ref, hv_ref, wet_ref, bet_ref,
                      wih_ref, whh_ref, bih_ref, bhh_ref,
                      wpn1_ref, bpn1_ref, wuv_ref,
                      h_ref, hp_ref, uwv_ref):
    accs = acc_ref[...]
    asum = accs[0] + accs[1]
    t = asum[:, :GF]
    den = asum[:, GF:GF + 1]
    rec = 1.0 / (den + 1e-12)
    ctx = _elu(jnp.dot(t * rec, wet_ref[...], preferred_element_type=_F32)
               + (den * rec) * bet_ref[...])
    hv = hv_ref[...]
    h = jnp.maximum(_gru(ctx, hv, wih_ref[...], whh_ref[...],
                         bih_ref[...], bhh_ref[...]), 0.0)
    h_ref[...] = h
    hp_ref[...] = (jnp.dot(h, wpn1_ref[...], preferred_element_type=_F32)
                   + bpn1_ref[...])
    uwv_ref[...] = jnp.dot(h, wuv_ref[...], preferred_element_type=_F32)


def tc_l1_update_body(acc_ref, h_ref, wih_ref, whh_ref, bih_ref, bhh_ref,
                      wpn0_ref, bpn0_ref, wpn1_ref, bpn1_ref,
                      wc2_ref, bc2_ref,
                      h2_ref, hvp0_ref, hvp1_ref, c2_ref):
    accs = acc_ref[...]
    asum = accs[0] + accs[1]
    t = asum[:, :GF]
    den = asum[:, GF:GF + 1]
    ctx = _elu(t / (den + 1e-12))
    h = h_ref[...]
    h2 = jnp.maximum(_gru(ctx, h, wih_ref[...], whh_ref[...],
                          bih_ref[...], bhh_ref[...]), 0.0)
    h2_ref[...] = h2
    hvp0_ref[...] = (jnp.dot(h2, wpn0_ref[...], preferred_element_type=_F32)
                     + bpn0_ref[...])
    hvp1_ref[...] = (jnp.dot(h2, wpn1_ref[...], preferred_element_type=_F32)
                     + bpn1_ref[...])
    c2_ref[...] = (jnp.dot(h2, wc2_ref[...], preferred_element_type=_F32)
                   + bc2_ref[...])


def tc_readout_body(h2_ref, hvp0_ref, hvp1_ref, c2_ref, gidf_ref,
                    wca0_ref, wca1_ref,
                    wih0_ref, whh0_ref, bih0_ref, bhh0_ref,
                    wih1_ref, whh1_ref, bih1_ref, bhh1_ref,
                    out_ref):
    nblk = V // BN
    giota = lax.broadcasted_iota(jnp.int32, (G, BN), 0).astype(_F32)

    def onehot(vb):
        gb = gidf_ref[pl.ds(vb, 1), :]
        return (giota == gb).astype(_F32)

    g = jnp.zeros((G, GF), _F32)
    for vb in range(nblk):
        g = g + jnp.dot(onehot(vb), h2_ref[pl.ds(vb * BN, BN), :],
                        preferred_element_type=_F32)

    for r in range(2):
        wca = (wca0_ref, wca1_ref)[r][...]
        hvp_ref = (hvp0_ref, hvp1_ref)[r]
        rg = jnp.maximum(g, 0.0)
        s1 = jnp.dot(rg, wca, preferred_element_type=_F32)
        tacc = jnp.zeros((G, GF), _F32)
        dacc = jnp.zeros((G, 8), _F32)
        for vb in range(nblk):
            oh = onehot(vb)
            s1n = lax.dot_general(oh, s1, (((0,), (0,)), ((), ())),
                                  preferred_element_type=_F32)
            c2b = c2_ref[pl.ds(vb * BN, BN), r * GF:r * GF + 1]
            w = jnp.exp(_lrelu(s1n[:, 0:1] + c2b))
            hvpb = hvp_ref[pl.ds(vb * BN, BN), :]
            tacc = tacc + jnp.dot(oh, w * hvpb,
                                  preferred_element_type=_F32)
            dacc = dacc + jnp.dot(oh, jnp.broadcast_to(w, (BN, 8)),
                                  preferred_element_type=_F32)
        ctx = _elu(tacc / (dacc[:, 0:1] + 1e-12))
        wih = (wih0_ref, wih1_ref)[r][...]
        whh = (whh0_ref, whh1_ref)[r][...]
        bih = (bih0_ref, bih1_ref)[r][...]
        bhh = (bhh0_ref, bhh1_ref)[r][...]
        g = jnp.maximum(_gru(ctx, g, wih, whh, bih, bhh), 0.0)
    out_ref[...] = g


_SC_MESH = dict(core_axis_name="c", subcore_axis_name="s",
                num_cores=NC, num_subcores=NS)


def sc_gc_body(p_hbm, eb_hbm, q_hbm, wd_hbm, src_hbm, dst_hbm, zero_hbm,
               out_hbm, srcall, dstall, prow0, prow1, eb0, eb1, msg0, msg1,
               db0, db1, qv, wdv, acc, gsem0, gsem1):
    cid = lax.axis_index("c")
    sid = lax.axis_index("s")
    wid = cid * NS + sid
    prow = (prow0, prow1)
    ebr = (eb0, eb1)
    msg = (msg0, msg1)
    db = (db0, db1)
    gsem = (gsem0, gsem1)

    pltpu.sync_copy(zero_hbm.at[pl.ds(sid * VPT, VPT)],
                    acc.at[pl.ds(sid * VPT, VPT)])
    pltpu.sync_copy(q_hbm, qv)
    pltpu.sync_copy(wd_hbm, wdv)
    pltpu.sync_copy(src_hbm.at[pl.ds(wid * EPW, EPW)], srcall)
    pltpu.sync_copy(dst_hbm.at[pl.ds(wid * EPW, EPW)], dstall)
    wd = [wdv[pl.ds(k * L, L)] for k in range(GF // L)]
    b2 = wdv[pl.ds(GF, L)][0]
    plsc.subcore_barrier()

    def issue(c, b):
        pltpu.async_copy(p_hbm.at[srcall.at[pl.ds(c * CHUNK, CHUNK)]],
                         prow[b], gsem[b])
        pltpu.async_copy(eb_hbm.at[pl.ds(wid * EPW + c * CHUNK, CHUNK)],
                         ebr[b], gsem[b])

    def wait_gather(b):
        pltpu.make_async_copy(eb_hbm.at[pl.ds(0, CHUNK)], prow[b],
                              gsem[b]).wait()
        pltpu.make_async_copy(eb_hbm.at[pl.ds(0, CHUNK)], ebr[b],
                              gsem[b]).wait()

    def process(c, b):
        wait_gather(b)
        for k in range(CHUNK // L):
            db[b][pl.ds(k * L, L)] = dstall[pl.ds(c * CHUNK + k * L, L)]

        def group_body(g, _):
            dv = db[b][pl.ds(g * L, L)]
            qd = plsc.load_gather(qv, [dv])
            for e in range(L):
                i = g * L + e
                hrows = []
                t = jnp.zeros((L,), _F32)
                for k in range(GF // L):
                    s = prow[b][i, pl.ds(k * L, L)] + ebr[b][i, pl.ds(k * L, L)]
                    hk = jnp.maximum(s, 0.01 * s)
                    hrows.append(hk)
                    t = t + hk * wd[k]
                lg = qd[e] + jnp.sum(t) + b2
                lg = jnp.maximum(lg, 0.01 * lg)
                w = jnp.exp(jnp.full((L,), lg, _F32))
                for k in range(GF // L):
                    msg[b][i, pl.ds(k * L, L)] = hrows[k] * w
                msg[b][i, pl.ds(GF, L)] = w
            return 0

        lax.fori_loop(0, CHUNK // L, group_body, 0)
        pltpu.sync_copy(msg[b], acc.at[db[b]], add=True)

    issue(0, 0)
    issue(1, 1)

    def pair_body(c2, _):
        for b in range(2):
            c = 2 * c2 + b
            process(c, b)

            @pl.when(c + 2 < NCH)
            def _():
                issue(c + 2, b)
        return 0

    lax.fori_loop(0, NCH // 2, pair_body, 0)
    process(NCH - 1, 0)
    plsc.subcore_barrier()
    pltpu.sync_copy(acc.at[pl.ds(sid * VPT, VPT)],
                    out_hbm.at[cid].at[pl.ds(sid * VPT, VPT)])


def sc_l1_body(hp_hbm, u_hbm, wv_hbm, b_hbm, src_hbm, dst_hbm, zero_hbm,
               out_hbm, srcall, dstall, hp0, hp1, msg0, msg1,
               db0, db1, uv, wvv, bv, acc, gsem0, gsem1):
    cid = lax.axis_index("c")
    sid = lax.axis_index("s")
    wid = cid * NS + sid
    hpr = (hp0, hp1)
    msg = (msg0, msg1)
    db = (db0, db1)
    gsem = (gsem0, gsem1)

    pltpu.sync_copy(zero_hbm.at[pl.ds(sid * VPT, VPT)],
                    acc.at[pl.ds(sid * VPT, VPT)])
    pltpu.sync_copy(u_hbm, uv)
    pltpu.sync_copy(wv_hbm, wvv)
    pltpu.sync_copy(b_hbm, bv)
    pltpu.sync_copy(src_hbm.at[pl.ds(wid * EPW, EPW)], srcall)
    pltpu.sync_copy(dst_hbm.at[pl.ds(wid * EPW, EPW)], dstall)
    bl = bv[...][0]
    plsc.subcore_barrier()

    def issue(c, b):
        pltpu.async_copy(hp_hbm.at[srcall.at[pl.ds(c * CHUNK, CHUNK)]],
                         hpr[b], gsem[b])

    def process(c, b):
        pltpu.make_async_copy(hp_hbm.at[pl.ds(0, CHUNK)], hpr[b],
                              gsem[b]).wait()
        for k in range(CHUNK // L):
            db[b][pl.ds(k * L, L)] = dstall[pl.ds(c * CHUNK + k * L, L)]

        def group_body(g, _):
            dv = db[b][pl.ds(g * L, L)]
            sv = srcall[pl.ds(c * CHUNK + g * L, L)]
            lg = plsc.load_gather(uv, [dv]) + plsc.load_gather(wvv, [sv]) + bl
            lg = jnp.maximum(lg, 0.01 * lg)
            wvec = jnp.exp(lg)
            for e in range(L):
                i = g * L + e
                w = jnp.full((L,), wvec[e], _F32)
                for k in range(GF // L):
                    msg[b][i, pl.ds(k * L, L)] = hpr[b][i, pl.ds(k * L, L)] * w
                msg[b][i, pl.ds(GF, L)] = w
            return 0

        lax.fori_loop(0, CHUNK // L, group_body, 0)
        pltpu.sync_copy(msg[b], acc.at[db[b]], add=True)

    issue(0, 0)
    issue(1, 1)

    def pair_body(c2, _):
        for b in range(2):
            c = 2 * c2 + b
            process(c, b)

            @pl.when(c + 2 < NCH)
            def _():
                issue(c + 2, b)
        return 0

    lax.fori_loop(0, NCH // 2, pair_body, 0)
    process(NCH - 1, 0)
    plsc.subcore_barrier()
    pltpu.sync_copy(acc.at[pl.ds(sid * VPT, VPT)],
                    out_hbm.at[cid].at[pl.ds(sid * VPT, VPT)])


def _full_spec(shape):
    nd = len(shape)
    return pl.BlockSpec(shape, lambda i, _n=nd: (0,) * _n)


def _call_tc_prep(nf, wpn, bpn, wa, wcb):
    return pl.pallas_call(
        tc_prep_body,
        grid=(V // BV,),
        in_specs=[
            pl.BlockSpec((BV, NF), lambda i: (i, 0)),
            _full_spec(wpn.shape), _full_spec(bpn.shape),
            _full_spec(wa.shape), _full_spec(wcb.shape),
        ],
        out_specs=[
            pl.BlockSpec((BV, GF), lambda i: (i, 0)),
            pl.BlockSpec((BV, GF), lambda i: (i, 0)),
            pl.BlockSpec((BV, 128), lambda i: (i, 0)),
        ],
        out_shape=[
            jax.ShapeDtypeStruct((V, GF), _F32),
            jax.ShapeDtypeStruct((V, GF), _F32),
            jax.ShapeDtypeStruct((V, 128), _F32),
        ],
    )(nf, wpn, bpn, wa, wcb)


def _call_tc_eb(ef, wb, bpe1):
    return pl.pallas_call(
        tc_eb_body,
        grid=(E // BE,),
        in_specs=[
            pl.BlockSpec((BE, EF), lambda i: (i, 0)),
            _full_spec(wb.shape), _full_spec(bpe1.shape),
        ],
        out_specs=pl.BlockSpec((BE, GF), lambda i: (i, 0)),
        out_shape=jax.ShapeDtypeStruct((E, GF), _F32),
    )(ef, wb, bpe1)


def _call_tc_gc_update(acc, hv, wet, bet, wih, whh, bih, bhh,
                       wpn1, bpn1, wuv):
    return pl.pallas_call(
        tc_gc_update_body,
        grid=(V // BV,),
        in_specs=[
            pl.BlockSpec((NC, BV, ACC_W), lambda i: (0, i, 0)),
            pl.BlockSpec((BV, GF), lambda i: (i, 0)),
            _full_spec(wet.shape), _full_spec(bet.shape),
            _full_spec(wih.shape), _full_spec(whh.shape),
            _full_spec(bih.shape), _full_spec(bhh.shape),
            _full_spec(wpn1.shape), _full_spec(bpn1.shape),
            _full_spec(wuv.shape),
        ],
        out_specs=[
            pl.BlockSpec((BV, GF), lambda i: (i, 0)),
            pl.BlockSpec((BV, GF), lambda i: (i, 0)),
            pl.BlockSpec((BV, 128), lambda i: (i, 0)),
        ],
        out_shape=[
            jax.ShapeDtypeStruct((V, GF), _F32),
            jax.ShapeDtypeStruct((V, GF), _F32),
            jax.ShapeDtypeStruct((V, 128), _F32),
        ],
    )(acc, hv, wet, bet, wih, whh, bih, bhh, wpn1, bpn1, wuv)


def _call_tc_l1_update(acc, h, wih, whh, bih, bhh,
                       wpn0, bpn0, wpn1, bpn1, wc2, bc2):
    return pl.pallas_call(
        tc_l1_update_body,
        grid=(V // BV,),
        in_specs=[
            pl.BlockSpec((NC, BV, ACC_W), lambda i: (0, i, 0)),
            pl.BlockSpec((BV, GF), lambda i: (i, 0)),
            _full_spec(wih.shape), _full_spec(whh.shape),
            _full_spec(bih.shape), _full_spec(bhh.shape),
            _full_spec(wpn0.shape), _full_spec(bpn0.shape),
            _full_spec(wpn1.shape), _full_spec(bpn1.shape),
            _full_spec(wc2.shape), _full_spec(bc2.shape),
        ],
        out_specs=[
            pl.BlockSpec((BV, GF), lambda i: (i, 0)),
            pl.BlockSpec((BV, GF), lambda i: (i, 0)),
            pl.BlockSpec((BV, GF), lambda i: (i, 0)),
            pl.BlockSpec((BV, 128), lambda i: (i, 0)),
        ],
        out_shape=[
            jax.ShapeDtypeStruct((V, GF), _F32),
            jax.ShapeDtypeStruct((V, GF), _F32),
            jax.ShapeDtypeStruct((V, GF), _F32),
            jax.ShapeDtypeStruct((V, 128), _F32),
        ],
    )(acc, h, wih, whh, bih, bhh, wpn0, bpn0, wpn1, bpn1, wc2, bc2)


def _call_tc_readout(h2, hvp0, hvp1, c2, gidf, wca0, wca1,
                     wih0, whh0, bih0, bhh0, wih1, whh1, bih1, bhh1):
    return pl.pallas_call(
        tc_readout_body,
        out_shape=jax.ShapeDtypeStruct((G, GF), _F32),
    )(h2, hvp0, hvp1, c2, gidf, wca0, wca1,
      wih0, whh0, bih0, bhh0, wih1, whh1, bih1, bhh1)


def _call_sc_gc(p, eb, q, wdpack, src, dst, zeros):
    f = functools.partial(
        pl.kernel,
        out_type=jax.ShapeDtypeStruct((NC, VP, ACC_W), _F32),
        mesh=plsc.VectorSubcoreMesh(**_SC_MESH),
        compiler_params=pltpu.CompilerParams(needs_layout_passes=False, use_tc_tiling_on_sc=False),
        scratch_types=[
            pltpu.VMEM((EPW,), jnp.int32),
            pltpu.VMEM((EPW,), jnp.int32),
            pltpu.VMEM((CHUNK, GF), _F32),
            pltpu.VMEM((CHUNK, GF), _F32),
            pltpu.VMEM((CHUNK, GF), _F32),
            pltpu.VMEM((CHUNK, GF), _F32),
            pltpu.VMEM((CHUNK, ACC_W), _F32),
            pltpu.VMEM((CHUNK, ACC_W), _F32),
            pltpu.VMEM((CHUNK,), jnp.int32),
            pltpu.VMEM((CHUNK,), jnp.int32),
            pltpu.VMEM((V,), _F32),
            pltpu.VMEM((ACC_W,), _F32),
            pltpu.VMEM_SHARED((VP, ACC_W), _F32),
            pltpu.SemaphoreType.DMA,
            pltpu.SemaphoreType.DMA,
        ],
    )(sc_gc_body)
    return f(p, eb, q, wdpack, src, dst, zeros)


def _call_sc_l1(hp, u, wv, bpack, src, dst, zeros):
    f = functools.partial(
        pl.kernel,
        out_type=jax.ShapeDtypeStruct((NC, VP, ACC_W), _F32),
        mesh=plsc.VectorSubcoreMesh(**_SC_MESH),
        compiler_params=pltpu.CompilerParams(needs_layout_passes=False, use_tc_tiling_on_sc=False),
        scratch_types=[
            pltpu.VMEM((EPW,), jnp.int32),
            pltpu.VMEM((EPW,), jnp.int32),
            pltpu.VMEM((CHUNK, GF), _F32),
            pltpu.VMEM((CHUNK, GF), _F32),
            pltpu.VMEM((CHUNK, ACC_W), _F32),
            pltpu.VMEM((CHUNK, ACC_W), _F32),
            pltpu.VMEM((CHUNK,), jnp.int32),
            pltpu.VMEM((CHUNK,), jnp.int32),
            pltpu.VMEM((V,), _F32),
            pltpu.VMEM((V,), _F32),
            pltpu.VMEM((L,), _F32),
            pltpu.VMEM_SHARED((VP, ACC_W), _F32),
            pltpu.SemaphoreType.DMA,
            pltpu.SemaphoreType.DMA,
        ],
    )(sc_l1_body)
    return f(hp, u, wv, bpack, src, dst, zeros)


def kernel(node_feats, edge_feats, params, edge_index, node_graph_ids):
    p_ = params
    src = edge_index[0]
    dst = edge_index[1]

    wpn = p_["gc_pn"]["W"]
    bpn = p_["gc_pn"]["b"].reshape(1, GF)
    wpe1 = p_["gc_pe1"]["W"]
    wa = wpe1[:NF]
    wb = wpe1[NF:]
    bpe1 = p_["gc_pe1"]["b"].reshape(1, GF)
    wpe2 = p_["gc_pe2"]["W"][:, 0]
    bpe2 = p_["gc_pe2"]["b"][0]
    wc = wpe2[:GF]
    wd = wpe2[GF:]
    wcb = jnp.broadcast_to(wc[:, None], (GF, 128))
    wdpack = jnp.zeros((ACC_W,), _F32).at[:GF].set(wd).at[GF].set(bpe2)

    zeros_acc = jnp.zeros((VP, ACC_W), _F32)

    hv, p, qpad = _call_tc_prep(node_feats, wpn, bpn, wa, wcb)
    q = qpad[:, 0]
    eb = _call_tc_eb(edge_feats, wb, bpe1)

    acc_gc = _call_sc_gc(p, eb, q, wdpack, src, dst, zeros_acc)

    wet = p_["gc_et"]["W"]
    bet = p_["gc_et"]["b"].reshape(1, GF)
    g_gru = p_["gc_gru"]
    wl1 = p_["l1_pe"]["W"][:, 0]
    bl1 = p_["l1_pe"]["b"][0]
    wuv = jnp.concatenate([
        jnp.broadcast_to(wl1[:GF, None], (GF, 64)),
        jnp.broadcast_to(wl1[GF:, None], (GF, 64)),
    ], axis=1)
    h, hp, uwv = _call_tc_gc_update(
        acc_gc, hv, wet, bet,
        g_gru["Wih"].T, g_gru["Whh"].T,
        g_gru["bih"].reshape(1, 3 * GF), g_gru["bhh"].reshape(1, 3 * GF),
        p_["l1_pn"]["W"], p_["l1_pn"]["b"].reshape(1, GF), wuv)
    u = uwv[:, 0]
    wv = uwv[:, 64]
    bpack = jnp.full((L,), bl1, _F32)

    acc_l1 = _call_sc_l1(hp, u, wv, bpack, src, dst, zeros_acc)

    l_gru = p_["l1_gru"]
    wc0 = p_["r0_cl"]["W"][:, 0]
    bc0 = p_["r0_cl"]["b"][0]
    wc1 = p_["r1_cl"]["W"][:, 0]
    bc1 = p_["r1_cl"]["b"][0]
    wc2 = jnp.concatenate([
        jnp.broadcast_to(wc0[GF:, None], (GF, 64)),
        jnp.broadcast_to(wc1[GF:, None], (GF, 64)),
    ], axis=1)
    bc2 = jnp.concatenate([jnp.full((1, 64), bc0, _F32),
                           jnp.full((1, 64), bc1, _F32)], axis=1)
    h2, hvp0, hvp1, c2 = _call_tc_l1_update(
        acc_l1, h,
        l_gru["Wih"].T, l_gru["Whh"].T,
        l_gru["bih"].reshape(1, 3 * GF), l_gru["bhh"].reshape(1, 3 * GF),
        p_["r0_pn"]["W"], p_["r0_pn"]["b"].reshape(1, GF),
        p_["r1_pn"]["W"], p_["r1_pn"]["b"].reshape(1, GF),
        wc2, bc2)

    gidf = node_graph_ids.astype(_F32).reshape(V // BN, BN)
    wca0 = jnp.broadcast_to(wc0[:GF, None], (GF, 8))
    wca1 = jnp.broadcast_to(wc1[:GF, None], (GF, 8))
    r0, r1 = p_["r0_gru"], p_["r1_gru"]
    out = _call_tc_readout(
        h2, hvp0, hvp1, c2, gidf, wca0, wca1,
        r0["Wih"].T, r0["Whh"].T,
        r0["bih"].reshape(1, 3 * GF), r0["bhh"].reshape(1, 3 * GF),
        r1["Wih"].T, r1["Whh"].T,
        r1["bih"].reshape(1, 3 * GF), r1["bhh"].reshape(1, 3 * GF))
    return out


# speedup vs baseline: 13.0041x; 1.0707x over previous
import functools

import jax
import jax.numpy as jnp
from jax import lax
from jax.experimental import pallas as pl
from jax.experimental.pallas import tpu as pltpu
from jax.experimental.pallas import tpu_sc as plsc

V, E, NF, EF, GF, G = 10000, 320000, 128, 16, 64, 256

NC, NS, L = 2, 16, 16
NW = NC * NS
EPW = E // NW
CHUNK = 80
NCH = EPW // CHUNK
ACC_W = 80
VP = 10240
VPT = VP // NS

BV = 2000
BE = 8000
BN = 1000

_F32 = jnp.float32


def _lrelu(x):
    return jnp.maximum(x, 0.01 * x)


def _elu(x):
    return jnp.where(x > 0, x, jnp.exp(jnp.minimum(x, 0.0)) - 1.0)


def _sigmoid(x):
    return 1.0 / (1.0 + jnp.exp(-x))


def _gru(x, h, wih_t, whh_t, bih, bhh):
    gi = jnp.dot(x, wih_t, preferred_element_type=_F32) + bih
    gh = jnp.dot(h, whh_t, preferred_element_type=_F32) + bhh
    r = _sigmoid(gi[:, 0:GF] + gh[:, 0:GF])
    z = _sigmoid(gi[:, GF:2 * GF] + gh[:, GF:2 * GF])
    n = jnp.tanh(gi[:, 2 * GF:] + r * gh[:, 2 * GF:])
    return (1.0 - z) * n + z * h


def tc_prep_body(nf_ref, wpn_ref, bpn_ref, wa_ref, wcb_ref,
                 hv_ref, p_ref, q_ref):
    nf = nf_ref[...]
    hv = _lrelu(jnp.dot(nf, wpn_ref[...], preferred_element_type=_F32)
                + bpn_ref[...])
    hv_ref[...] = hv
    p_ref[...] = jnp.dot(nf, wa_ref[...], preferred_element_type=_F32)
    q_ref[...] = jnp.dot(hv, wcb_ref[...], preferred_element_type=_F32)


def tc_eb_body(ef_ref, wb_ref, bpe1_ref, eb_ref):
    eb_ref[...] = (jnp.dot(ef_ref[...], wb_ref[...],
                           preferred_element_type=_F32) + bpe1_ref[...])


def tc_gc_update_body(acc_ref, hv_ref, wet_ref, bet_ref,
                      wih_ref, whh_ref, bih_ref, bhh_ref,
                      wpn1_ref, bpn1_ref, wuv_ref,
                      h_ref, hp_ref, uwv_ref):
    accs = acc_ref[...]
    asum = accs[0] + accs[1]
    t = asum[:, :GF]
    den = asum[:, GF:GF + 1]
    rec = 1.0 / (den + 1e-12)
    ctx = _elu(jnp.dot(t * rec, wet_ref[...], preferred_element_type=_F32)
               + (den * rec) * bet_ref[...])
    hv = hv_ref[...]
    h = jnp.maximum(_gru(ctx, hv, wih_ref[...], whh_ref[...],
                         bih_ref[...], bhh_ref[...]), 0.0)
    h_ref[...] = h
    hp_ref[...] = (jnp.dot(h, wpn1_ref[...], preferred_element_type=_F32)
                   + bpn1_ref[...])
    uwv_ref[...] = jnp.dot(h, wuv_ref[...], preferred_element_type=_F32)


def tc_fin_body(acc_ref, h_ref, gidf_ref,
                wih_ref, whh_ref, bih_ref, bhh_ref,
                wpn0_ref, bpn0_ref, wpn1_ref, bpn1_ref,
                wc80_ref, bc80_ref, wc81_ref, bc81_ref,
                wca0_ref, wca1_ref,
                wih0_ref, whh0_ref, bih0_ref, bhh0_ref,
                wih1_ref, whh1_ref, bih1_ref, bhh1_ref,
                out_ref, h2s_ref):
    nblk = V // BN
    giota = lax.broadcasted_iota(jnp.int32, (G, BN), 0).astype(_F32)

    def onehot(vb):
        gb = gidf_ref[pl.ds(vb, 1), :]
        return (giota == gb).astype(_F32)

    wih = wih_ref[...]
    whh = whh_ref[...]
    bih = bih_ref[...]
    bhh = bhh_ref[...]
    g = jnp.zeros((G, GF), _F32)
    for vb in range(nblk):
        acb = (acc_ref[0, pl.ds(vb * BN, BN), :]
               + acc_ref[1, pl.ds(vb * BN, BN), :])
        ctx = _elu(acb[:, :GF] / (acb[:, GF:GF + 1] + 1e-12))
        h2b = jnp.maximum(_gru(ctx, h_ref[pl.ds(vb * BN, BN), :],
                               wih, whh, bih, bhh), 0.0)
        h2s_ref[pl.ds(vb * BN, BN), :] = h2b
        g = g + jnp.dot(onehot(vb), h2b, preferred_element_type=_F32)

    for r in range(2):
        wca = (wca0_ref, wca1_ref)[r][...]
        wpn = (wpn0_ref, wpn1_ref)[r][...]
        bpn = (bpn0_ref, bpn1_ref)[r][...]
        wc8 = (wc80_ref, wc81_ref)[r][...]
        bc8 = (bc80_ref, bc81_ref)[r][...]
        rg = jnp.maximum(g, 0.0)
        s1 = jnp.dot(rg, wca, preferred_element_type=_F32)
        tacc = jnp.zeros((G, GF), _F32)
        dacc = jnp.zeros((G, 8), _F32)
        for vb in range(nblk):
            oh = onehot(vb)
            h2b = h2s_ref[pl.ds(vb * BN, BN), :]
            s1n = lax.dot_general(oh, s1, (((0,), (0,)), ((), ())),
                                  preferred_element_type=_F32)
            c2b = (jnp.dot(h2b, wc8, preferred_element_type=_F32)
                   + bc8)[:, 0:1]
            w = jnp.exp(_lrelu(s1n[:, 0:1] + c2b))
            hvpb = jnp.dot(h2b, wpn, preferred_element_type=_F32) + bpn
            tacc = tacc + jnp.dot(oh, w * hvpb,
                                  preferred_element_type=_F32)
            dacc = dacc + jnp.dot(oh, jnp.broadcast_to(w, (BN, 8)),
                                  preferred_element_type=_F32)
        ctx = _elu(tacc / (dacc[:, 0:1] + 1e-12))
        wihr = (wih0_ref, wih1_ref)[r][...]
        whhr = (whh0_ref, whh1_ref)[r][...]
        bihr = (bih0_ref, bih1_ref)[r][...]
        bhhr = (bhh0_ref, bhh1_ref)[r][...]
        g = jnp.maximum(_gru(ctx, g, wihr, whhr, bihr, bhhr), 0.0)
    out_ref[...] = g


_SC_MESH = dict(core_axis_name="c", subcore_axis_name="s",
                num_cores=NC, num_subcores=NS)


def sc_gc_body(p_hbm, eb_hbm, q_hbm, wd_hbm, src_hbm, dst_hbm, zero_hbm,
               out_hbm, srcall, dstall, prow0, prow1, eb0, eb1, msg0, msg1,
               db0, db1, qv, wdv, acc, gsem0, gsem1, ssem0, ssem1):
    cid = lax.axis_index("c")
    sid = lax.axis_index("s")
    wid = cid * NS + sid
    prow = (prow0, prow1)
    ebr = (eb0, eb1)
    msg = (msg0, msg1)
    db = (db0, db1)
    gsem = (gsem0, gsem1)
    ssem = (ssem0, ssem1)

    pltpu.sync_copy(zero_hbm.at[pl.ds(sid * VPT, VPT)],
                    acc.at[pl.ds(sid * VPT, VPT)])
    pltpu.sync_copy(q_hbm, qv)
    pltpu.sync_copy(wd_hbm, wdv)
    pltpu.sync_copy(src_hbm.at[pl.ds(wid * EPW, EPW)], srcall)
    pltpu.sync_copy(dst_hbm.at[pl.ds(wid * EPW, EPW)], dstall)
    wd = [wdv[pl.ds(k * L, L)] for k in range(GF // L)]
    b2 = wdv[pl.ds(GF, L)][0]
    plsc.subcore_barrier()

    def issue(c, b):
        pltpu.async_copy(p_hbm.at[srcall.at[pl.ds(c * CHUNK, CHUNK)]],
                         prow[b], gsem[b])
        pltpu.async_copy(eb_hbm.at[pl.ds(wid * EPW + c * CHUNK, CHUNK)],
                         ebr[b], gsem[b])

    def wait_gather(b):
        pltpu.make_async_copy(eb_hbm.at[pl.ds(0, CHUNK)], prow[b],
                              gsem[b]).wait()
        pltpu.make_async_copy(eb_hbm.at[pl.ds(0, CHUNK)], ebr[b],
                              gsem[b]).wait()

    def process(c, b):
        wait_gather(b)

        @pl.when(c >= 2)
        def _():
            pltpu.make_async_copy(zero_hbm.at[pl.ds(0, CHUNK)], msg[b],
                                  ssem[b]).wait()

        for k in range(CHUNK // L):
            db[b][pl.ds(k * L, L)] = dstall[pl.ds(c * CHUNK + k * L, L)]

        def group_body(g, _):
            dv = db[b][pl.ds(g * L, L)]
            qd = plsc.load_gather(qv, [dv])
            for e in range(L):
                i = g * L + e
                hrows = []
                t = jnp.zeros((L,), _F32)
                for k in range(GF // L):
                    s = prow[b][i, pl.ds(k * L, L)] + ebr[b][i, pl.ds(k * L, L)]
                    hk = jnp.maximum(s, 0.01 * s)
                    hrows.append(hk)
                    t = t + hk * wd[k]
                lg = qd[e] + jnp.sum(t) + b2
                lg = jnp.maximum(lg, 0.01 * lg)
                w = jnp.exp(jnp.full((L,), lg, _F32))
                for k in range(GF // L):
                    msg[b][i, pl.ds(k * L, L)] = hrows[k] * w
                msg[b][i, pl.ds(GF, L)] = w
            return 0

        lax.fori_loop(0, CHUNK // L, group_body, 0)
        pltpu.async_copy(msg[b], acc.at[db[b]], ssem[b], add=True)

    issue(0, 0)
    issue(1, 1)

    def pair_body(c2, _):
        for b in range(2):
            c = 2 * c2 + b
            process(c, b)

            @pl.when(c + 2 < NCH)
            def _():
                issue(c + 2, b)
        return 0

    lax.fori_loop(0, NCH // 2, pair_body, 0)
    process(NCH - 1, 0)
    pltpu.make_async_copy(zero_hbm.at[pl.ds(0, CHUNK)], msg[0], ssem[0]).wait()
    pltpu.make_async_copy(zero_hbm.at[pl.ds(0, CHUNK)], msg[1], ssem[1]).wait()
    plsc.subcore_barrier()
    pltpu.sync_copy(acc.at[pl.ds(sid * VPT, VPT)],
                    out_hbm.at[cid].at[pl.ds(sid * VPT, VPT)])


def sc_l1_body(hp_hbm, u_hbm, wv_hbm, b_hbm, src_hbm, dst_hbm, zero_hbm,
               out_hbm, srcall, dstall, hp0, hp1, msg0, msg1,
               db0, db1, uv, wvv, bv, acc, gsem0, gsem1, ssem0, ssem1):
    cid = lax.axis_index("c")
    sid = lax.axis_index("s")
    wid = cid * NS + sid
    hpr = (hp0, hp1)
    msg = (msg0, msg1)
    db = (db0, db1)
    gsem = (gsem0, gsem1)
    ssem = (ssem0, ssem1)

    pltpu.sync_copy(zero_hbm.at[pl.ds(sid * VPT, VPT)],
                    acc.at[pl.ds(sid * VPT, VPT)])
    pltpu.sync_copy(u_hbm, uv)
    pltpu.sync_copy(wv_hbm, wvv)
    pltpu.sync_copy(b_hbm, bv)
    pltpu.sync_copy(src_hbm.at[pl.ds(wid * EPW, EPW)], srcall)
    pltpu.sync_copy(dst_hbm.at[pl.ds(wid * EPW, EPW)], dstall)
    bl = bv[...][0]
    plsc.subcore_barrier()

    def issue(c, b):
        pltpu.async_copy(hp_hbm.at[srcall.at[pl.ds(c * CHUNK, CHUNK)]],
                         hpr[b], gsem[b])

    def process(c, b):
        pltpu.make_async_copy(hp_hbm.at[pl.ds(0, CHUNK)], hpr[b],
                              gsem[b]).wait()

        @pl.when(c >= 2)
        def _():
            pltpu.make_async_copy(zero_hbm.at[pl.ds(0, CHUNK)], msg[b],
                                  ssem[b]).wait()

        for k in range(CHUNK // L):
            db[b][pl.ds(k * L, L)] = dstall[pl.ds(c * CHUNK + k * L, L)]

        def group_body(g, _):
            dv = db[b][pl.ds(g * L, L)]
            sv = srcall[pl.ds(c * CHUNK + g * L, L)]
            lg = plsc.load_gather(uv, [dv]) + plsc.load_gather(wvv, [sv]) + bl
            lg = jnp.maximum(lg, 0.01 * lg)
            wvec = jnp.exp(lg)
            for e in range(L):
                i = g * L + e
                w = jnp.full((L,), wvec[e], _F32)
                for k in range(GF // L):
                    msg[b][i, pl.ds(k * L, L)] = hpr[b][i, pl.ds(k * L, L)] * w
                msg[b][i, pl.ds(GF, L)] = w
            return 0

        lax.fori_loop(0, CHUNK // L, group_body, 0)
        pltpu.async_copy(msg[b], acc.at[db[b]], ssem[b], add=True)

    issue(0, 0)
    issue(1, 1)

    def pair_body(c2, _):
        for b in range(2):
            c = 2 * c2 + b
            process(c, b)

            @pl.when(c + 2 < NCH)
            def _():
                issue(c + 2, b)
        return 0

    lax.fori_loop(0, NCH // 2, pair_body, 0)
    process(NCH - 1, 0)
    pltpu.make_async_copy(zero_hbm.at[pl.ds(0, CHUNK)], msg[0], ssem[0]).wait()
    pltpu.make_async_copy(zero_hbm.at[pl.ds(0, CHUNK)], msg[1], ssem[1]).wait()
    plsc.subcore_barrier()
    pltpu.sync_copy(acc.at[pl.ds(sid * VPT, VPT)],
                    out_hbm.at[cid].at[pl.ds(sid * VPT, VPT)])


def _full_spec(shape):
    nd = len(shape)
    return pl.BlockSpec(shape, lambda i, _n=nd: (0,) * _n)


def _call_tc_prep(nf, wpn, bpn, wa, wcb):
    return pl.pallas_call(
        tc_prep_body,
        grid=(V // BV,),
        in_specs=[
            pl.BlockSpec((BV, NF), lambda i: (i, 0)),
            _full_spec(wpn.shape), _full_spec(bpn.shape),
            _full_spec(wa.shape), _full_spec(wcb.shape),
        ],
        out_specs=[
            pl.BlockSpec((BV, GF), lambda i: (i, 0)),
            pl.BlockSpec((BV, GF), lambda i: (i, 0)),
            pl.BlockSpec((BV, 8), lambda i: (i, 0)),
        ],
        out_shape=[
            jax.ShapeDtypeStruct((V, GF), _F32),
            jax.ShapeDtypeStruct((V, GF), _F32),
            jax.ShapeDtypeStruct((V, 8), _F32),
        ],
    )(nf, wpn, bpn, wa, wcb)


def _call_tc_eb(ef, wb, bpe1):
    return pl.pallas_call(
        tc_eb_body,
        grid=(E // BE,),
        in_specs=[
            pl.BlockSpec((BE, EF), lambda i: (i, 0)),
            _full_spec(wb.shape), _full_spec(bpe1.shape),
        ],
        out_specs=pl.BlockSpec((BE, GF), lambda i: (i, 0)),
        out_shape=jax.ShapeDtypeStruct((E, GF), _F32),
    )(ef, wb, bpe1)


def _call_tc_gc_update(acc, hv, wet, bet, wih, whh, bih, bhh,
                       wpn1, bpn1, wuv):
    return pl.pallas_call(
        tc_gc_update_body,
        grid=(V // BV,),
        in_specs=[
            pl.BlockSpec((NC, BV, ACC_W), lambda i: (0, i, 0)),
            pl.BlockSpec((BV, GF), lambda i: (i, 0)),
            _full_spec(wet.shape), _full_spec(bet.shape),
            _full_spec(wih.shape), _full_spec(whh.shape),
            _full_spec(bih.shape), _full_spec(bhh.shape),
            _full_spec(wpn1.shape), _full_spec(bpn1.shape),
            _full_spec(wuv.shape),
        ],
        out_specs=[
            pl.BlockSpec((BV, GF), lambda i: (i, 0)),
            pl.BlockSpec((BV, GF), lambda i: (i, 0)),
            pl.BlockSpec((BV, 16), lambda i: (i, 0)),
        ],
        out_shape=[
            jax.ShapeDtypeStruct((V, GF), _F32),
            jax.ShapeDtypeStruct((V, GF), _F32),
            jax.ShapeDtypeStruct((V, 16), _F32),
        ],
    )(acc, hv, wet, bet, wih, whh, bih, bhh, wpn1, bpn1, wuv)


def _call_tc_fin(acc, h, gidf, *weights):
    return pl.pallas_call(
        tc_fin_body,
        out_shape=jax.ShapeDtypeStruct((G, GF), _F32),
        scratch_shapes=[pltpu.VMEM((V, GF), _F32)],
    )(acc, h, gidf, *weights)


def _call_sc_gc(p, eb, q, wdpack, src, dst, zeros):
    f = functools.partial(
        pl.kernel,
        out_type=jax.ShapeDtypeStruct((NC, VP, ACC_W), _F32),
        mesh=plsc.VectorSubcoreMesh(**_SC_MESH),
        compiler_params=pltpu.CompilerParams(needs_layout_passes=False, use_tc_tiling_on_sc=False),
        scratch_types=[
            pltpu.VMEM((EPW,), jnp.int32),
            pltpu.VMEM((EPW,), jnp.int32),
            pltpu.VMEM((CHUNK, GF), _F32),
            pltpu.VMEM((CHUNK, GF), _F32),
            pltpu.VMEM((CHUNK, GF), _F32),
            pltpu.VMEM((CHUNK, GF), _F32),
            pltpu.VMEM((CHUNK, ACC_W), _F32),
            pltpu.VMEM((CHUNK, ACC_W), _F32),
            pltpu.VMEM((CHUNK,), jnp.int32),
            pltpu.VMEM((CHUNK,), jnp.int32),
            pltpu.VMEM((V,), _F32),
            pltpu.VMEM((ACC_W,), _F32),
            pltpu.VMEM_SHARED((VP, ACC_W), _F32),
            pltpu.SemaphoreType.DMA,
            pltpu.SemaphoreType.DMA,
            pltpu.SemaphoreType.DMA,
            pltpu.SemaphoreType.DMA,
        ],
    )(sc_gc_body)
    return f(p, eb, q, wdpack, src, dst, zeros)


def _call_sc_l1(hp, u, wv, bpack, src, dst, zeros):
    f = functools.partial(
        pl.kernel,
        out_type=jax.ShapeDtypeStruct((NC, VP, ACC_W), _F32),
        mesh=plsc.VectorSubcoreMesh(**_SC_MESH),
        compiler_params=pltpu.CompilerParams(needs_layout_passes=False, use_tc_tiling_on_sc=False),
        scratch_types=[
            pltpu.VMEM((EPW,), jnp.int32),
            pltpu.VMEM((EPW,), jnp.int32),
            pltpu.VMEM((CHUNK, GF), _F32),
            pltpu.VMEM((CHUNK, GF), _F32),
            pltpu.VMEM((CHUNK, ACC_W), _F32),
            pltpu.VMEM((CHUNK, ACC_W), _F32),
            pltpu.VMEM((CHUNK,), jnp.int32),
            pltpu.VMEM((CHUNK,), jnp.int32),
            pltpu.VMEM((V,), _F32),
            pltpu.VMEM((V,), _F32),
            pltpu.VMEM((L,), _F32),
            pltpu.VMEM_SHARED((VP, ACC_W), _F32),
            pltpu.SemaphoreType.DMA,
            pltpu.SemaphoreType.DMA,
            pltpu.SemaphoreType.DMA,
            pltpu.SemaphoreType.DMA,
        ],
    )(sc_l1_body)
    return f(hp, u, wv, bpack, src, dst, zeros)


def kernel(node_feats, edge_feats, params, edge_index, node_graph_ids):
    p_ = params
    src = edge_index[0]
    dst = edge_index[1]

    wpn = p_["gc_pn"]["W"]
    bpn = p_["gc_pn"]["b"].reshape(1, GF)
    wpe1 = p_["gc_pe1"]["W"]
    wa = wpe1[:NF]
    wb = wpe1[NF:]
    bpe1 = p_["gc_pe1"]["b"].reshape(1, GF)
    wpe2 = p_["gc_pe2"]["W"][:, 0]
    bpe2 = p_["gc_pe2"]["b"][0]
    wc = wpe2[:GF]
    wd = wpe2[GF:]
    wcb = jnp.broadcast_to(wc[:, None], (GF, 8))
    wdpack = jnp.zeros((ACC_W,), _F32).at[:GF].set(wd).at[GF].set(bpe2)

    zeros_acc = jnp.zeros((VP, ACC_W), _F32)

    hv, p, qpad = _call_tc_prep(node_feats, wpn, bpn, wa, wcb)
    q = qpad[:, 0]
    eb = _call_tc_eb(edge_feats, wb, bpe1)

    acc_gc = _call_sc_gc(p, eb, q, wdpack, src, dst, zeros_acc)

    wet = p_["gc_et"]["W"]
    bet = p_["gc_et"]["b"].reshape(1, GF)
    g_gru = p_["gc_gru"]
    wl1 = p_["l1_pe"]["W"][:, 0]
    bl1 = p_["l1_pe"]["b"][0]
    wuv = jnp.concatenate([
        jnp.broadcast_to(wl1[:GF, None], (GF, 8)),
        jnp.broadcast_to(wl1[GF:, None], (GF, 8)),
    ], axis=1)
    h, hp, uwv = _call_tc_gc_update(
        acc_gc, hv, wet, bet,
        g_gru["Wih"].T, g_gru["Whh"].T,
        g_gru["bih"].reshape(1, 3 * GF), g_gru["bhh"].reshape(1, 3 * GF),
        p_["l1_pn"]["W"], p_["l1_pn"]["b"].reshape(1, GF), wuv)
    u = uwv[:, 0]
    wv = uwv[:, 8]
    bpack = jnp.full((L,), bl1, _F32)

    acc_l1 = _call_sc_l1(hp, u, wv, bpack, src, dst, zeros_acc)

    l_gru = p_["l1_gru"]
    wc0 = p_["r0_cl"]["W"][:, 0]
    bc0 = p_["r0_cl"]["b"][0]
    wc1 = p_["r1_cl"]["W"][:, 0]
    bc1 = p_["r1_cl"]["b"][0]
    gidf = node_graph_ids.astype(_F32).reshape(V // BN, BN)
    wca0 = jnp.broadcast_to(wc0[:GF, None], (GF, 8))
    wca1 = jnp.broadcast_to(wc1[:GF, None], (GF, 8))
    wc80 = jnp.broadcast_to(wc0[GF:, None], (GF, 8))
    wc81 = jnp.broadcast_to(wc1[GF:, None], (GF, 8))
    bc80 = jnp.full((1, 8), bc0, _F32)
    bc81 = jnp.full((1, 8), bc1, _F32)
    r0, r1 = p_["r0_gru"], p_["r1_gru"]
    out = _call_tc_fin(
        acc_l1, h, gidf,
        l_gru["Wih"].T, l_gru["Whh"].T,
        l_gru["bih"].reshape(1, 3 * GF), l_gru["bhh"].reshape(1, 3 * GF),
        p_["r0_pn"]["W"], p_["r0_pn"]["b"].reshape(1, GF),
        p_["r1_pn"]["W"], p_["r1_pn"]["b"].reshape(1, GF),
        wc80, bc80, wc81, bc81, wca0, wca1,
        r0["Wih"].T, r0["Whh"].T,
        r0["bih"].reshape(1, 3 * GF), r0["bhh"].reshape(1, 3 * GF),
        r1["Wih"].T, r1["Whh"].T,
        r1["bih"].reshape(1, 3 * GF), r1["bhh"].reshape(1, 3 * GF))
    return out
